# baseline (speedup 1.0000x reference)
import functools

import jax
import jax.numpy as jnp
from jax import lax
from jax.experimental import pallas as pl
from jax.experimental.pallas import tpu as pltpu
from jax.experimental.pallas import tpu_sc as plsc

B, N, C = 4, 4096, 256
S, K = 1024, 32
C_COV, C_OUT = 64, 512


def _fps_kernel(xyzT_ref, idx_ref, newT_ref, dists_ref, far_ref):
    x = xyzT_ref[:, 0, :]
    y = xyzT_ref[:, 1, :]
    z = xyzT_ref[:, 2, :]
    iota = jax.lax.broadcasted_iota(jnp.int32, (B, N), 1)
    iota_s = jax.lax.broadcasted_iota(jnp.int32, (B, S), 1)

    dists_ref[...] = jnp.full((B, N), 1e10, jnp.float32)
    far_ref[...] = jnp.zeros((B, 1), jnp.int32)
    idx_ref[...] = jnp.zeros((B, S), jnp.int32)
    newT_ref[...] = jnp.zeros((B, 3, S), jnp.float32)

    def body(i, carry):
        far = far_ref[...]
        sloti = (iota_s == i).astype(jnp.int32)
        slotf = sloti.astype(jnp.float32)
        idx_ref[...] = idx_ref[...] + sloti * far
        sel = iota == far
        cx = jnp.max(jnp.where(sel, x, -jnp.inf), axis=1, keepdims=True)
        cy = jnp.max(jnp.where(sel, y, -jnp.inf), axis=1, keepdims=True)
        cz = jnp.max(jnp.where(sel, z, -jnp.inf), axis=1, keepdims=True)
        newT_ref[:, 0, :] = newT_ref[:, 0, :] + slotf * cx
        newT_ref[:, 1, :] = newT_ref[:, 1, :] + slotf * cy
        newT_ref[:, 2, :] = newT_ref[:, 2, :] + slotf * cz
        d = (x - cx) ** 2 + (y - cy) ** 2 + (z - cz) ** 2
        dists = jnp.minimum(dists_ref[...], d)
        dists_ref[...] = dists
        m = jnp.max(dists, axis=1, keepdims=True)
        cand = jnp.where(dists == m, iota, N)
        far_ref[...] = jnp.min(cand, axis=1, keepdims=True)
        return carry

    jax.lax.fori_loop(0, S, body, 0)


def _run_fps(xyzT):
    return pl.pallas_call(
        _fps_kernel,
        out_shape=(
            jax.ShapeDtypeStruct((B, S), jnp.int32),
            jax.ShapeDtypeStruct((B, 3, S), jnp.float32),
        ),
        scratch_shapes=[
            pltpu.VMEM((B, N), jnp.float32),
            pltpu.VMEM((B, 1), jnp.int32),
        ],
    )(xyzT)


QB = 256


def _knn_kernel(bias, qT_ref, xyzT_ref, idx_ref, s12_ref):
    q = qT_ref[0]
    xT = xyzT_ref[0]
    qq = jnp.sum(q * q, axis=1, keepdims=True)
    xx = jnp.sum(xT * xT, axis=0, keepdims=True)
    qx = jax.lax.dot_general(q, xT, (((1,), (0,)), ((), ())),
                             preferred_element_type=jnp.float32)
    d2 = qq - 2.0 * qx + xx
    iota = jax.lax.broadcasted_iota(jnp.int32, (QB, N), 1)

    cur = d2
    iota_k = jax.lax.broadcasted_iota(jnp.int32, (QB, K), 1)
    idx_acc = jnp.zeros((QB, K), jnp.int32)
    for k in range(K):
        mval = jnp.min(cur, axis=1, keepdims=True)
        cand = jnp.where(cur == mval, iota, N)
        pick = jnp.min(cand, axis=1, keepdims=True)
        idx_acc = idx_acc + (iota_k == k).astype(jnp.int32) * pick
        cur = jnp.where(cand == pick, jnp.inf, cur)
    w = (cur == jnp.inf).astype(jnp.float32)
    idx_ref[0] = idx_acc + bias

    x = xT[0:1, :]
    y = xT[1:2, :]
    z = xT[2:3, :]
    rows = jnp.concatenate(
        [x, y, z, x * x, x * y, x * z, y * x, y * y, y * z, z * x, z * y, z * z],
        axis=0)
    s12 = jax.lax.dot_general(w, rows, (((1,), (1,)), ((), ())),
                              preferred_element_type=jnp.float32)
    s12_ref[0] = s12


def _run_knn_batch(newT, xyzT, b):
    return pl.pallas_call(
        functools.partial(_knn_kernel, b * N),
        grid=(S // QB,),
        in_specs=[
            pl.BlockSpec((1, QB, 3), lambda s: (b, s, 0)),
            pl.BlockSpec((1, 3, N), lambda s: (b, 0, 0)),
        ],
        out_specs=[
            pl.BlockSpec((1, QB, K), lambda s: (0, s, 0)),
            pl.BlockSpec((1, QB, 12), lambda s: (0, s, 0)),
        ],
        out_shape=[
            jax.ShapeDtypeStruct((1, S, K), jnp.int32),
            jax.ShapeDtypeStruct((1, S, 12), jnp.float32),
        ],
    )(newT, xyzT)


try:
    _info = plsc.get_sparse_core_info()
    _NC, _NS = _info.num_cores, _info.num_subcores
except Exception:
    _NC, _NS = 2, 16
_NW = _NC * _NS
SP = S // _NW
_LANES = 16


def _sc_gather_max_kernel(idx_hbm, fT_hbm, out_hbm, idx_v, rows_v, out_v, sem):
    wid = lax.axis_index("s") * _NC + lax.axis_index("c")
    base = wid * SP
    pltpu.sync_copy(idx_hbm.at[pl.ds(base, SP)], idx_v)

    def per_pair(j, carry):
        pltpu.async_copy(fT_hbm.at[idx_v.at[j]], rows_v, sem).wait()
        for c in range(C // _LANES):
            sl = pl.ds(c * _LANES, _LANES)
            acc = rows_v[0, sl]
            acc = lax.fori_loop(
                1, K, lambda k, a: jnp.maximum(a, rows_v[k, sl]), acc)
            out_v[j, sl] = acc
        return carry

    lax.fori_loop(0, SP, per_pair, 0)
    pltpu.sync_copy(out_v, out_hbm.at[pl.ds(base, SP)])


@functools.lru_cache(maxsize=None)
def _sc_gather_max_fn():
    return functools.partial(
        pl.kernel,
        out_type=jax.ShapeDtypeStruct((S, C), jnp.float32),
        mesh=plsc.VectorSubcoreMesh(core_axis_name="c", subcore_axis_name="s"),
        scratch_types=[
            pltpu.VMEM((SP, K), jnp.int32),
            pltpu.VMEM((K, C), jnp.float32),
            pltpu.VMEM((SP, C), jnp.float32),
            pltpu.SemaphoreType.DMA,
        ],
    )(_sc_gather_max_kernel)


def _sc_gather_max(idx, fT2):
    return _sc_gather_max_fn()(idx, fT2)


def _head_kernel(s12_ref, fmax_ref, wc_ref, bc_ref, wf_ref, bf_ref, out_ref):
    s12 = s12_ref[0]
    mu = s12[:, 0:3] / float(K)
    s2 = s12[:, 3:12] / float(K)
    mumu = (mu[:, :, None] * mu[:, None, :]).reshape(S, 9)
    cf = s2 - mumu
    wc = wc_ref[...]
    fcov = jax.lax.dot_general(cf, wc, (((1,), (1,)), ((), ())),
                               preferred_element_type=jnp.float32)
    fcov = jnp.maximum(fcov + bc_ref[...][None, :], 0.0)
    fmax = fmax_ref[0]
    fc = jnp.concatenate([fmax, fcov], axis=1)
    wf = wf_ref[...]
    out = jax.lax.dot_general(wf, fc, (((1,), (1,)), ((), ())),
                              preferred_element_type=jnp.float32)
    out_ref[0] = jnp.maximum(out + bf_ref[...][:, None], 0.0)


def _run_head(s12, fmax, W_cov, b_cov, W_f, b_f):
    nb = s12.shape[0]
    return pl.pallas_call(
        _head_kernel,
        grid=(nb,),
        in_specs=[
            pl.BlockSpec((1, S, 12), lambda b: (b, 0, 0)),
            pl.BlockSpec((1, S, C), lambda b: (b, 0, 0)),
            pl.BlockSpec((C_COV, 9), lambda b: (0, 0)),
            pl.BlockSpec((C_COV,), lambda b: (0,)),
            pl.BlockSpec((C_OUT, C + C_COV), lambda b: (0, 0)),
            pl.BlockSpec((C_OUT,), lambda b: (0,)),
        ],
        out_specs=pl.BlockSpec((1, C_OUT, S), lambda b: (b, 0, 0)),
        out_shape=jax.ShapeDtypeStruct((nb, C_OUT, S), jnp.float32),
    )(s12, fmax, W_cov, b_cov, W_f, b_f)


@jax.jit
def kernel(f, xyz, W_cov, b_cov, W_f, b_f):
    xyzT = xyz.transpose(0, 2, 1)
    idx_fps, newT = _run_fps(xyzT)
    xyz_new = newT.transpose(0, 2, 1)
    newTq = xyz_new
    fT2 = f.transpose(0, 2, 1).reshape(B * N, C)
    outs = []
    for b in range(B):
        idx_b, s12_b = _run_knn_batch(newTq, xyzT, b)
        fmax_b = _sc_gather_max(idx_b[0], fT2)[None]
        outs.append(_run_head(s12_b, fmax_b, W_cov, b_cov, W_f, b_f))
    out = jnp.concatenate(outs, axis=0)
    return (out, xyz_new)


# speedup vs baseline: 1.0172x; 1.0172x over previous
import functools

import jax
import jax.numpy as jnp
from jax import lax
from jax.experimental import pallas as pl
from jax.experimental.pallas import tpu as pltpu
from jax.experimental.pallas import tpu_sc as plsc

B, N, C = 4, 4096, 256
S, K = 1024, 32
C_COV, C_OUT = 64, 512


def _fps_kernel(xyzT_ref, idx_ref, newT_ref, dists_ref, far_ref):
    x = xyzT_ref[:, 0, :]
    y = xyzT_ref[:, 1, :]
    z = xyzT_ref[:, 2, :]
    iota = jax.lax.broadcasted_iota(jnp.int32, (B, N), 1)
    iota_s = jax.lax.broadcasted_iota(jnp.int32, (B, S), 1)

    dists_ref[...] = jnp.full((B, N), 1e10, jnp.float32)
    far_ref[...] = jnp.zeros((B, 1), jnp.int32)
    idx_ref[...] = jnp.zeros((B, S), jnp.int32)
    newT_ref[...] = jnp.zeros((B, 3, S), jnp.float32)

    def body(i, carry):
        far = far_ref[...]
        sloti = (iota_s == i).astype(jnp.int32)
        slotf = sloti.astype(jnp.float32)
        idx_ref[...] = idx_ref[...] + sloti * far
        sel = iota == far
        cx = jnp.max(jnp.where(sel, x, -jnp.inf), axis=1, keepdims=True)
        cy = jnp.max(jnp.where(sel, y, -jnp.inf), axis=1, keepdims=True)
        cz = jnp.max(jnp.where(sel, z, -jnp.inf), axis=1, keepdims=True)
        newT_ref[:, 0, :] = newT_ref[:, 0, :] + slotf * cx
        newT_ref[:, 1, :] = newT_ref[:, 1, :] + slotf * cy
        newT_ref[:, 2, :] = newT_ref[:, 2, :] + slotf * cz
        d = (x - cx) ** 2 + (y - cy) ** 2 + (z - cz) ** 2
        dists = jnp.minimum(dists_ref[...], d)
        dists_ref[...] = dists
        m = jnp.max(dists, axis=1, keepdims=True)
        cand = jnp.where(dists == m, iota, N)
        far_ref[...] = jnp.min(cand, axis=1, keepdims=True)
        return carry

    jax.lax.fori_loop(0, S, body, 0)


def _run_fps(xyzT):
    return pl.pallas_call(
        _fps_kernel,
        out_shape=(
            jax.ShapeDtypeStruct((B, S), jnp.int32),
            jax.ShapeDtypeStruct((B, 3, S), jnp.float32),
        ),
        scratch_shapes=[
            pltpu.VMEM((B, N), jnp.float32),
            pltpu.VMEM((B, 1), jnp.int32),
        ],
    )(xyzT)


QB = 256


def _knn_kernel(bias, qT_ref, xyzT_ref, idx_ref, s12_ref):
    q = qT_ref[0]
    xT = xyzT_ref[0]
    qq = jnp.sum(q * q, axis=1, keepdims=True)
    xx = jnp.sum(xT * xT, axis=0, keepdims=True)
    qx = jax.lax.dot_general(q, xT, (((1,), (0,)), ((), ())),
                             preferred_element_type=jnp.float32)
    d2 = qq - 2.0 * qx + xx
    iota = jax.lax.broadcasted_iota(jnp.int32, (QB, N), 1)

    cur = d2
    iota_k = jax.lax.broadcasted_iota(jnp.int32, (QB, K), 1)
    idx_acc = jnp.zeros((QB, K), jnp.int32)
    for k in range(K):
        mval = jnp.min(cur, axis=1, keepdims=True)
        cand = jnp.where(cur == mval, iota, N)
        pick = jnp.min(cand, axis=1, keepdims=True)
        idx_acc = idx_acc + (iota_k == k).astype(jnp.int32) * pick
        cur = jnp.where(cand == pick, jnp.inf, cur)
    w = (cur == jnp.inf).astype(jnp.float32)
    idx_ref[0] = idx_acc + bias

    x = xT[0:1, :]
    y = xT[1:2, :]
    z = xT[2:3, :]
    rows = jnp.concatenate(
        [x, y, z, x * x, x * y, x * z, y * x, y * y, y * z, z * x, z * y, z * z],
        axis=0)
    s12 = jax.lax.dot_general(w, rows, (((1,), (1,)), ((), ())),
                              preferred_element_type=jnp.float32)
    s12_ref[0] = s12


def _run_knn_batch(newT, xyzT, b):
    return pl.pallas_call(
        functools.partial(_knn_kernel, b * N),
        grid=(S // QB,),
        in_specs=[
            pl.BlockSpec((1, QB, 3), lambda s: (b, s, 0)),
            pl.BlockSpec((1, 3, N), lambda s: (b, 0, 0)),
        ],
        out_specs=[
            pl.BlockSpec((1, QB, K), lambda s: (0, s, 0)),
            pl.BlockSpec((1, QB, 12), lambda s: (0, s, 0)),
        ],
        out_shape=[
            jax.ShapeDtypeStruct((1, S, K), jnp.int32),
            jax.ShapeDtypeStruct((1, S, 12), jnp.float32),
        ],
    )(newT, xyzT)


try:
    _info = plsc.get_sparse_core_info()
    _NC, _NS = _info.num_cores, _info.num_subcores
except Exception:
    _NC, _NS = 2, 16
_NW = _NC * _NS
SP = S // _NW
_LANES = 16


def _sc_gather_max_kernel(idx_hbm, fT_hbm, out_hbm, idx_v, rows_v, out_v,
                          sem_a, sem_b):
    wid = lax.axis_index("s") * _NC + lax.axis_index("c")
    base = wid * SP
    pltpu.sync_copy(idx_hbm.at[pl.ds(base, SP)], idx_v)

    def reduce_one(buf, j):
        accs = [rows_v[buf, 0, pl.ds(c * _LANES, _LANES)]
                for c in range(C // _LANES)]
        for k in range(1, K):
            for c in range(C // _LANES):
                accs[c] = jnp.maximum(
                    accs[c], rows_v[buf, k, pl.ds(c * _LANES, _LANES)])
        for c in range(C // _LANES):
            out_v[j, pl.ds(c * _LANES, _LANES)] = accs[c]

    pltpu.async_copy(fT_hbm.at[idx_v.at[0]], rows_v.at[0], sem_a)

    def per_pair(t, carry):
        j0 = 2 * t
        j1 = 2 * t + 1
        pltpu.make_async_copy(fT_hbm.at[idx_v.at[j0]], rows_v.at[0],
                              sem_a).wait()
        pltpu.async_copy(fT_hbm.at[idx_v.at[j1]], rows_v.at[1], sem_b)
        reduce_one(0, j0)
        pltpu.make_async_copy(fT_hbm.at[idx_v.at[j1]], rows_v.at[1],
                              sem_b).wait()
        jn = jnp.minimum(j0 + 2, SP - 1)
        pltpu.async_copy(fT_hbm.at[idx_v.at[jn]], rows_v.at[0], sem_a)
        reduce_one(1, j1)
        return carry

    lax.fori_loop(0, SP // 2, per_pair, 0)
    pltpu.make_async_copy(fT_hbm.at[idx_v.at[SP - 1]], rows_v.at[0],
                          sem_a).wait()
    pltpu.sync_copy(out_v, out_hbm.at[pl.ds(base, SP)])


@functools.lru_cache(maxsize=None)
def _sc_gather_max_fn():
    return functools.partial(
        pl.kernel,
        out_type=jax.ShapeDtypeStruct((S, C), jnp.float32),
        mesh=plsc.VectorSubcoreMesh(core_axis_name="c", subcore_axis_name="s"),
        scratch_types=[
            pltpu.VMEM((SP, K), jnp.int32),
            pltpu.VMEM((2, K, C), jnp.float32),
            pltpu.VMEM((SP, C), jnp.float32),
            pltpu.SemaphoreType.DMA,
            pltpu.SemaphoreType.DMA,
        ],
    )(_sc_gather_max_kernel)


def _sc_gather_max(idx, fT2):
    return _sc_gather_max_fn()(idx, fT2)


def _head_kernel(s12_ref, fmax_ref, wc_ref, bc_ref, wf_ref, bf_ref, out_ref):
    s12 = s12_ref[0]
    mu = s12[:, 0:3] / float(K)
    s2 = s12[:, 3:12] / float(K)
    mumu = (mu[:, :, None] * mu[:, None, :]).reshape(S, 9)
    cf = s2 - mumu
    wc = wc_ref[...]
    fcov = jax.lax.dot_general(cf, wc, (((1,), (1,)), ((), ())),
                               preferred_element_type=jnp.float32)
    fcov = jnp.maximum(fcov + bc_ref[...][None, :], 0.0)
    fmax = fmax_ref[0]
    fc = jnp.concatenate([fmax, fcov], axis=1)
    wf = wf_ref[...]
    out = jax.lax.dot_general(wf, fc, (((1,), (1,)), ((), ())),
                              preferred_element_type=jnp.float32)
    out_ref[0] = jnp.maximum(out + bf_ref[...][:, None], 0.0)


def _run_head(s12, fmax, W_cov, b_cov, W_f, b_f):
    nb = s12.shape[0]
    return pl.pallas_call(
        _head_kernel,
        grid=(nb,),
        in_specs=[
            pl.BlockSpec((1, S, 12), lambda b: (b, 0, 0)),
            pl.BlockSpec((1, S, C), lambda b: (b, 0, 0)),
            pl.BlockSpec((C_COV, 9), lambda b: (0, 0)),
            pl.BlockSpec((C_COV,), lambda b: (0,)),
            pl.BlockSpec((C_OUT, C + C_COV), lambda b: (0, 0)),
            pl.BlockSpec((C_OUT,), lambda b: (0,)),
        ],
        out_specs=pl.BlockSpec((1, C_OUT, S), lambda b: (b, 0, 0)),
        out_shape=jax.ShapeDtypeStruct((nb, C_OUT, S), jnp.float32),
    )(s12, fmax, W_cov, b_cov, W_f, b_f)


@jax.jit
def kernel(f, xyz, W_cov, b_cov, W_f, b_f):
    xyzT = xyz.transpose(0, 2, 1)
    idx_fps, newT = _run_fps(xyzT)
    xyz_new = newT.transpose(0, 2, 1)
    newTq = xyz_new
    fT2 = f.transpose(0, 2, 1).reshape(B * N, C)
    outs = []
    for b in range(B):
        idx_b, s12_b = _run_knn_batch(newTq, xyzT, b)
        fmax_b = _sc_gather_max(idx_b[0], fT2)[None]
        outs.append(_run_head(s12_b, fmax_b, W_cov, b_cov, W_f, b_f))
    out = jnp.concatenate(outs, axis=0)
    return (out, xyz_new)
